# baseline (speedup 1.0000x reference)
import functools

import jax
import jax.numpy as jnp
from jax import lax
from jax.experimental import pallas as pl
from jax.experimental.pallas import tpu as pltpu
from jax.experimental.pallas import tpu_sc as plsc

NUM_FEATURES = 26
VOCAB = 100000
EMBED = 32
BATCH = 16384

NC = 2
NS = 16
NW = NC * NS
LANES = 16

VCHUNK = 25600
N_VCHUNKS = 4
SUB = VCHUNK // 4
ROWS_PER_F = N_VCHUNKS * SUB
VPAD_F = ROWS_PER_F * 4

B_W = BATCH // NW
IDX_SLICE = 128
GATHERS = B_W // IDX_SLICE

BCHUNK = 16384
N_BCHUNKS = BATCH // BCHUNK


def _eye(n):
    r = lax.broadcasted_iota(jnp.int32, (n, n), 0)
    c = lax.broadcasted_iota(jnp.int32, (n, n), 1)
    return jnp.where(r == c, jnp.float32(1), jnp.float32(0))


def _rows_body(i_ref, o_ref):
    ident = _eye(EMBED)
    for u in range(4):
        o_ref[:, pl.ds(EMBED * u, EMBED)] = lax.dot_general(
            i_ref[0, :, pl.ds(SUB * u, SUB)],
            ident,
            (((0,), (0,)), ((), ())),
            preferred_element_type=jnp.float32,
        )


def _out_body(i_ref, o_ref):
    ident = _eye(EMBED)
    q = BCHUNK // 4
    for j in range(4):
        o_ref[0, :, pl.ds(q * j, q)] = lax.dot_general(
            ident,
            i_ref[0, :, pl.ds(EMBED * j, EMBED)],
            (((1,), (1,)), ((), ())),
            preferred_element_type=jnp.float32,
        )


def _gather_body(xt, tab, out, idx_v, rows_v, gsem):
    wid = lax.axis_index("s") * NC + lax.axis_index("c")
    b0 = wid * B_W
    wpc = BCHUNK // B_W
    wpj = (BCHUNK // 4) // B_W
    chunk = wid // wpc
    rem = wid - chunk * wpc
    jj = rem // wpj
    off = (rem - jj * wpj) * B_W

    one = jnp.full((LANES,), 1, jnp.int32)
    zero = jnp.full((LANES,), 0, jnp.int32)

    def feature_body(f, carry):
        pltpu.sync_copy(xt.at[f, pl.ds(b0, B_W)], idx_v)
        for k in range(B_W // LANES):
            sl = pl.ds(k * LANES, LANES)
            v = idx_v[sl]
            c7 = zero
            for t in range(1, N_VCHUNKS):
                c7 = c7 + jnp.where(v >= t * VCHUNK, one, zero)
            w = v - c7 * VCHUNK
            u3 = (
                jnp.where(w >= SUB, one, zero)
                + jnp.where(w >= 2 * SUB, one, zero)
                + jnp.where(w >= 3 * SUB, one, zero)
            )
            idx_v[sl] = c7 * VCHUNK + (w - u3 * SUB) * 4 + u3
        copies = []
        for k in range(GATHERS):
            cp = pltpu.async_copy(
                tab.at[f].at[idx_v.at[pl.ds(k * IDX_SLICE, IDX_SLICE)]],
                rows_v.at[pl.ds(k * IDX_SLICE, IDX_SLICE)],
                gsem,
            )
            copies.append(cp)
        for cp in copies:
            cp.wait()
        pltpu.sync_copy(rows_v, out.at[f, chunk, pl.ds(off, B_W), jj, :])
        return carry

    lax.fori_loop(0, NUM_FEATURES, feature_body, 0)


def kernel(x, tables):
    xt = x.T
    tt = tables.transpose(0, 2, 1)

    t128 = pl.pallas_call(
        _rows_body,
        out_shape=jax.ShapeDtypeStruct((NUM_FEATURES * ROWS_PER_F, 128), jnp.float32),
        grid=(NUM_FEATURES, N_VCHUNKS),
        in_specs=[pl.BlockSpec((1, EMBED, VCHUNK), lambda f, c: (f, 0, c))],
        out_specs=pl.BlockSpec((SUB, 128), lambda f, c: (f * N_VCHUNKS + c, 0)),
        compiler_params=pltpu.CompilerParams(fuse_transposed_lhs_in_matmul=True),
    )(tt)

    gat = functools.partial(
        pl.kernel,
        out_type=jax.ShapeDtypeStruct(
            (NUM_FEATURES, N_BCHUNKS, BCHUNK // 4, 4, EMBED), jnp.float32
        ),
        mesh=plsc.VectorSubcoreMesh(core_axis_name="c", subcore_axis_name="s"),
        compiler_params=pltpu.CompilerParams(use_tc_tiling_on_sc=False),
        scratch_types=[
            pltpu.VMEM((B_W,), jnp.int32),
            pltpu.VMEM((B_W, EMBED), jnp.float32),
            pltpu.SemaphoreType.DMA,
        ],
    )(_gather_body)

    tab3 = t128.reshape(NUM_FEATURES, VPAD_F, EMBED)
    out_t = gat(xt, tab3)

    o3 = pl.pallas_call(
        _out_body,
        out_shape=jax.ShapeDtypeStruct((NUM_FEATURES, EMBED, BATCH), jnp.float32),
        grid=(NUM_FEATURES, N_BCHUNKS),
        in_specs=[pl.BlockSpec((1, BCHUNK // 4, 128), lambda f, c: (f, c, 0))],
        out_specs=pl.BlockSpec((1, EMBED, BCHUNK), lambda f, c: (f, 0, c)),
    )(out_t.reshape(NUM_FEATURES, BATCH // 4, 128))

    return o3.transpose(2, 0, 1)


# speedup vs baseline: 1.3438x; 1.3438x over previous
import functools

import jax
import jax.numpy as jnp
from jax import lax
from jax.experimental import pallas as pl
from jax.experimental.pallas import tpu as pltpu
from jax.experimental.pallas import tpu_sc as plsc

NUM_FEATURES = 26
VOCAB = 100000
EMBED = 32
BATCH = 16384

NC = 2
NS = 16
NW = NC * NS
LANES = 16

VCHUNK = 25600
N_VCHUNKS = 4
SUB = VCHUNK // 4
ROWS_PER_F = N_VCHUNKS * SUB
VPAD_F = ROWS_PER_F * 4

B_W = BATCH // NW
IDX_SLICE = 128
GATHERS = B_W // IDX_SLICE

BCHUNK = 8192
N_BCHUNKS = BATCH // BCHUNK


def _eye(n):
    r = lax.broadcasted_iota(jnp.int32, (n, n), 0)
    c = lax.broadcasted_iota(jnp.int32, (n, n), 1)
    return jnp.where(r == c, jnp.float32(1), jnp.float32(0))


def _rows_body(i_ref, o_ref):
    ident = _eye(EMBED)
    for u in range(4):
        o_ref[:, pl.ds(EMBED * u, EMBED)] = lax.dot_general(
            i_ref[0, :, pl.ds(SUB * u, SUB)],
            ident,
            (((0,), (0,)), ((), ())),
            preferred_element_type=jnp.float32,
        )


def _out_body(i_ref, o_ref):
    ident = _eye(EMBED)
    q = BCHUNK // 4
    for j in range(4):
        o_ref[0, :, pl.ds(q * j, q)] = lax.dot_general(
            ident,
            i_ref[0, :, pl.ds(EMBED * j, EMBED)],
            (((1,), (1,)), ((), ())),
            preferred_element_type=jnp.float32,
        )


def _gather_body(xt, tab, out, idx_v, rows_v, gsem):
    wid = lax.axis_index("s") * NC + lax.axis_index("c")
    b0 = wid * B_W
    wpc = BCHUNK // B_W
    wpj = (BCHUNK // 4) // B_W
    chunk = wid // wpc
    rem = wid - chunk * wpc
    jj = rem // wpj
    off = (rem - jj * wpj) * B_W

    one = jnp.full((LANES,), 1, jnp.int32)
    zero = jnp.full((LANES,), 0, jnp.int32)

    def feature_body(f, carry):
        pltpu.sync_copy(xt.at[f, pl.ds(b0, B_W)], idx_v)
        for k in range(B_W // LANES):
            sl = pl.ds(k * LANES, LANES)
            v = idx_v[sl]
            c7 = zero
            for t in range(1, N_VCHUNKS):
                c7 = c7 + jnp.where(v >= t * VCHUNK, one, zero)
            w = v - c7 * VCHUNK
            u3 = (
                jnp.where(w >= SUB, one, zero)
                + jnp.where(w >= 2 * SUB, one, zero)
                + jnp.where(w >= 3 * SUB, one, zero)
            )
            idx_v[sl] = c7 * VCHUNK + (w - u3 * SUB) * 4 + u3
        copies = []
        for k in range(GATHERS):
            cp = pltpu.async_copy(
                tab.at[f].at[idx_v.at[pl.ds(k * IDX_SLICE, IDX_SLICE)]],
                rows_v.at[pl.ds(k * IDX_SLICE, IDX_SLICE)],
                gsem,
            )
            copies.append(cp)
        for cp in copies:
            cp.wait()
        pltpu.sync_copy(rows_v, out.at[f, chunk, pl.ds(off, B_W), jj, :])
        return carry

    lax.fori_loop(0, NUM_FEATURES, feature_body, 0)


def kernel(x, tables):
    xt = x.T
    tt = tables.transpose(0, 2, 1)

    t128 = pl.pallas_call(
        _rows_body,
        out_shape=jax.ShapeDtypeStruct((NUM_FEATURES * ROWS_PER_F, 128), jnp.float32),
        grid=(NUM_FEATURES, N_VCHUNKS),
        in_specs=[pl.BlockSpec((1, EMBED, VCHUNK), lambda f, c: (f, 0, c))],
        out_specs=pl.BlockSpec((SUB, 128), lambda f, c: (f * N_VCHUNKS + c, 0)),
        compiler_params=pltpu.CompilerParams(fuse_transposed_lhs_in_matmul=True),
    )(tt)

    gat = functools.partial(
        pl.kernel,
        out_type=jax.ShapeDtypeStruct(
            (NUM_FEATURES, N_BCHUNKS, BCHUNK // 4, 4, EMBED), jnp.float32
        ),
        mesh=plsc.VectorSubcoreMesh(core_axis_name="c", subcore_axis_name="s"),
        compiler_params=pltpu.CompilerParams(use_tc_tiling_on_sc=False),
        scratch_types=[
            pltpu.VMEM((B_W,), jnp.int32),
            pltpu.VMEM((B_W, EMBED), jnp.float32),
            pltpu.SemaphoreType.DMA,
        ],
    )(_gather_body)

    tab3 = t128.reshape(NUM_FEATURES, VPAD_F, EMBED)
    out_t = gat(xt, tab3)

    o3 = pl.pallas_call(
        _out_body,
        out_shape=jax.ShapeDtypeStruct((NUM_FEATURES, EMBED, BATCH), jnp.float32),
        grid=(NUM_FEATURES, N_BCHUNKS),
        in_specs=[pl.BlockSpec((1, BCHUNK // 4, 128), lambda f, c: (f, c, 0))],
        out_specs=pl.BlockSpec((1, EMBED, BCHUNK), lambda f, c: (f, 0, c)),
    )(out_t.reshape(NUM_FEATURES, BATCH // 4, 128))

    return o3.transpose(2, 0, 1)
